# baseline (speedup 1.0000x reference)
import jax
import jax.numpy as jnp
from jax.experimental import pallas as pl
from jax.experimental.pallas import tpu as pltpu
from jax.experimental.pallas import tpu_sc as plsc

N = 10000
K = 32
D = 128
NK = N * K
P = 200
R = P * K
GW = 128
PAD = 327680


def _ln(x, g, b, eps=1e-5):
    m = jnp.mean(x, axis=-1, keepdims=True)
    v = jnp.mean((x - m) ** 2, axis=-1, keepdims=True)
    return (x - m) / jnp.sqrt(v + eps) * g + b


def _leaky(x):
    return jnp.where(x >= 0, x, 0.2 * x)


def _dot(x, w):
    return jnp.dot(x.astype(jnp.bfloat16), w.astype(jnp.bfloat16),
                   preferred_element_type=jnp.float32)


def _sc_gather(table, idx):
    mesh = plsc.VectorSubcoreMesh(core_axis_name="core", subcore_axis_name="subcore")

    @pl.kernel(
        out_type=jax.ShapeDtypeStruct((PAD, table.shape[1]), table.dtype),
        mesh=mesh,
    )
    def kern(tab_hbm, i_hbm, o_hbm):
        def body(i_vmem, o_vmem):
            pltpu.sync_copy(tab_hbm.at[i_vmem.at[0]], o_vmem)

        pltpu.emit_pipeline(
            body,
            grid=(PAD // GW,),
            in_specs=[pl.BlockSpec((1, GW), index_map=lambda i: (0, i))],
            out_specs=[pl.BlockSpec((GW, table.shape[1]), index_map=lambda i: (i, 0))],
            core_axis_name=("core", "subcore"),
            dimension_semantics=(pltpu.PARALLEL,),
        )(i_hbm, o_hbm)

    return kern(table, idx)


def _fused_body(geom_ref, s_ref,
                gW1_ref, gb1_ref, gg1_ref, gB1_ref,
                gW2_ref, gb2_ref, gg2_ref, gB2_ref,
                gW3_ref, gb3_ref,
                fW1a_ref, fW1b_ref, fb1_ref, fg1_ref, fB1_ref,
                fW2_ref, fb2_ref, fg2_ref, fB2_ref,
                fW3_ref, fb3_ref,
                out_ref):
    x = geom_ref[...]
    h = _leaky(_ln(_dot(x, gW1_ref[...]) + gb1_ref[...],
                   gg1_ref[...], gB1_ref[...]))
    h = _leaky(_ln(_dot(h, gW2_ref[...]) + gb2_ref[...],
                   gg2_ref[...], gB2_ref[...]))
    go = _dot(h, gW3_ref[...]) + gb3_ref[...]

    rel_term = _dot(s_ref[...], fW1b_ref[...])
    a1 = _dot(go, fW1a_ref[...]) + rel_term + fb1_ref[...]
    h = _leaky(_ln(a1, fg1_ref[...], fB1_ref[...]))
    h = _leaky(_ln(_dot(h, fW2_ref[...]) + fb2_ref[...],
                   fg2_ref[...], fB2_ref[...]))
    ff = _dot(h, fW3_ref[...]) + fb3_ref[...]

    out_ref[...] = jnp.mean(ff.reshape(P, K, D), axis=1)


def _row2(v):
    return v.reshape(1, -1)


def kernel(point_features, geom_features, neighbor_idxs,
           gW1, gb1, gg1, gB1, gW2, gb2, gg2, gB2, gW3, gb3,
           fW1, fb1, fg1, fB1, fW2, fb2, fg2, fB2, fW3, fb3):
    pf = point_features.reshape(N, D)
    fW1a, fW1b = fW1[:D], fW1[D:]

    idx = neighbor_idxs.reshape(-1).astype(jnp.int32)
    pad_rows = (jnp.arange(PAD - NK, dtype=jnp.int32) * 13) % N
    idx = jnp.concatenate([idx, pad_rows]).reshape(1, PAD)
    s = _sc_gather(pf, idx)

    geom = geom_features.reshape(NK, 4)

    wspec = lambda shape: pl.BlockSpec(shape, lambda i: (0, 0))
    in_specs = [
        pl.BlockSpec((R, 4), lambda i: (i, 0)),
        pl.BlockSpec((R, D), lambda i: (i, 0)),
        wspec((4, 64)), wspec((1, 64)), wspec((1, 64)), wspec((1, 64)),
        wspec((64, 128)), wspec((1, 128)), wspec((1, 128)), wspec((1, 128)),
        wspec((128, D)), wspec((1, D)),
        wspec((D, 64)), wspec((D, 64)), wspec((1, 64)), wspec((1, 64)), wspec((1, 64)),
        wspec((64, 128)), wspec((1, 128)), wspec((1, 128)), wspec((1, 128)),
        wspec((128, D)), wspec((1, D)),
    ]
    out = pl.pallas_call(
        _fused_body,
        grid=(N // P,),
        in_specs=in_specs,
        out_specs=pl.BlockSpec((P, D), lambda i: (i, 0)),
        out_shape=jax.ShapeDtypeStruct((N, D), jnp.float32),
    )(geom, s,
      gW1, _row2(gb1), _row2(gg1), _row2(gB1),
      gW2, _row2(gb2), _row2(gg2), _row2(gB2),
      gW3, _row2(gb3),
      fW1a, fW1b, _row2(fb1), _row2(fg1), _row2(fB1),
      fW2, _row2(fb2), _row2(fg2), _row2(fB2),
      fW3, _row2(fb3))

    return out.reshape(1, N, D)


# speedup vs baseline: 1.0197x; 1.0197x over previous
import jax
import jax.numpy as jnp
from jax.experimental import pallas as pl
from jax.experimental.pallas import tpu as pltpu
from jax.experimental.pallas import tpu_sc as plsc

N = 10000
K = 32
D = 128
NK = N * K
P = 200
R = P * K
GW = 128
PAD = 327680


def _ln(x, g, b, eps=1e-5):
    m = jnp.mean(x, axis=-1, keepdims=True)
    v = jnp.mean((x - m) ** 2, axis=-1, keepdims=True)
    return (x - m) / jnp.sqrt(v + eps) * g + b


def _leaky(x):
    return jnp.where(x >= 0, x, 0.2 * x)


def _dot(x, w):
    return jnp.dot(x, w, preferred_element_type=jnp.float32)


def _sc_gather(table, idx):
    mesh = plsc.VectorSubcoreMesh(core_axis_name="core", subcore_axis_name="subcore")

    @pl.kernel(
        out_type=jax.ShapeDtypeStruct((PAD, table.shape[1]), table.dtype),
        mesh=mesh,
    )
    def kern(tab_hbm, i_hbm, o_hbm):
        def body(i_vmem, o_vmem):
            pltpu.sync_copy(tab_hbm.at[i_vmem.at[0]], o_vmem)

        pltpu.emit_pipeline(
            body,
            grid=(PAD // GW,),
            in_specs=[pl.BlockSpec((1, GW), index_map=lambda i: (0, i))],
            out_specs=[pl.BlockSpec((GW, table.shape[1]), index_map=lambda i: (i, 0))],
            core_axis_name=("core", "subcore"),
            dimension_semantics=(pltpu.PARALLEL,),
        )(i_hbm, o_hbm)

    return kern(table, idx)


def _fused_body(geom_ref, s_ref,
                gW1_ref, gb1_ref, gg1_ref, gB1_ref,
                gW2_ref, gb2_ref, gg2_ref, gB2_ref,
                gW3_ref, gb3_ref,
                fW1a_ref, fW1b_ref, fb1_ref, fg1_ref, fB1_ref,
                fW2_ref, fb2_ref, fg2_ref, fB2_ref,
                fW3_ref, fb3_ref,
                out_ref):
    x = geom_ref[...]
    h = _leaky(_ln(_dot(x, gW1_ref[...]) + gb1_ref[...],
                   gg1_ref[...], gB1_ref[...]))
    h = _leaky(_ln(_dot(h, gW2_ref[...]) + gb2_ref[...],
                   gg2_ref[...], gB2_ref[...]))
    go = _dot(h, gW3_ref[...]) + gb3_ref[...]

    rel_term = _dot(s_ref[...], fW1b_ref[...])
    a1 = _dot(go, fW1a_ref[...]) + rel_term + fb1_ref[...]
    h = _leaky(_ln(a1, fg1_ref[...], fB1_ref[...]))
    h = _leaky(_ln(_dot(h, fW2_ref[...]) + fb2_ref[...],
                   fg2_ref[...], fB2_ref[...]))
    ff = _dot(h, fW3_ref[...]) + fb3_ref[...]

    out_ref[...] = jnp.mean(ff.reshape(P, K, D), axis=1)


def _row2(v):
    return v.reshape(1, -1)


def kernel(point_features, geom_features, neighbor_idxs,
           gW1, gb1, gg1, gB1, gW2, gb2, gg2, gB2, gW3, gb3,
           fW1, fb1, fg1, fB1, fW2, fb2, fg2, fB2, fW3, fb3):
    pf = point_features.reshape(N, D)
    fW1a, fW1b = fW1[:D], fW1[D:]

    idx = neighbor_idxs.reshape(-1).astype(jnp.int32)
    pad_rows = (jnp.arange(PAD - NK, dtype=jnp.int32) * 13) % N
    idx = jnp.concatenate([idx, pad_rows]).reshape(1, PAD)
    s = _sc_gather(pf, idx)

    geom = geom_features.reshape(NK, 4)

    wspec = lambda shape: pl.BlockSpec(shape, lambda i: (0, 0))
    in_specs = [
        pl.BlockSpec((R, 4), lambda i: (i, 0)),
        pl.BlockSpec((R, D), lambda i: (i, 0)),
        wspec((4, 64)), wspec((1, 64)), wspec((1, 64)), wspec((1, 64)),
        wspec((64, 128)), wspec((1, 128)), wspec((1, 128)), wspec((1, 128)),
        wspec((128, D)), wspec((1, D)),
        wspec((D, 64)), wspec((D, 64)), wspec((1, 64)), wspec((1, 64)), wspec((1, 64)),
        wspec((64, 128)), wspec((1, 128)), wspec((1, 128)), wspec((1, 128)),
        wspec((128, D)), wspec((1, D)),
    ]
    out = pl.pallas_call(
        _fused_body,
        grid=(N // P,),
        in_specs=in_specs,
        out_specs=pl.BlockSpec((P, D), lambda i: (i, 0)),
        out_shape=jax.ShapeDtypeStruct((N, D), jnp.float32),
    )(geom, s,
      gW1, _row2(gb1), _row2(gg1), _row2(gB1),
      gW2, _row2(gb2), _row2(gg2), _row2(gB2),
      gW3, _row2(gb3),
      fW1a, fW1b, _row2(fb1), _row2(fg1), _row2(fB1),
      fW2, _row2(fb2), _row2(fg2), _row2(fB2),
      fW3, _row2(fb3))

    return out.reshape(1, N, D)


# speedup vs baseline: 1.3374x; 1.3116x over previous
import jax
import jax.numpy as jnp
from jax.experimental import pallas as pl
from jax.experimental.pallas import tpu as pltpu
from jax.experimental.pallas import tpu_sc as plsc

N = 10000
K = 32
D = 128
NK = N * K
P = 200
R = P * K
GW = 128
PAD = 327680


def _ln(x, g, b, jmat, eps=1e-5):
    m = jnp.dot(x, jmat, preferred_element_type=jnp.float32)
    d = x - m
    v = jnp.dot(d * d, jmat, preferred_element_type=jnp.float32)
    s = g * jax.lax.rsqrt(v + eps)
    return d * s + b


def _leaky(x):
    return jnp.maximum(x, 0.2 * x)


def _dot(x, w):
    return jnp.dot(x, w, preferred_element_type=jnp.float32)


def _sc_gather(table, idx):
    mesh = plsc.VectorSubcoreMesh(core_axis_name="core", subcore_axis_name="subcore")

    @pl.kernel(
        out_type=jax.ShapeDtypeStruct((PAD, table.shape[1]), table.dtype),
        mesh=mesh,
    )
    def kern(tab_hbm, i_hbm, o_hbm):
        def body(i_vmem, o_vmem):
            pltpu.sync_copy(tab_hbm.at[i_vmem.at[0]], o_vmem)

        pltpu.emit_pipeline(
            body,
            grid=(PAD // GW,),
            in_specs=[pl.BlockSpec((1, GW), index_map=lambda i: (0, i))],
            out_specs=[pl.BlockSpec((GW, table.shape[1]), index_map=lambda i: (i, 0))],
            core_axis_name=("core", "subcore"),
            dimension_semantics=(pltpu.PARALLEL,),
        )(i_hbm, o_hbm)

    return kern(table, idx)


def _fused_body(geom_ref, s_ref,
                gW1_ref, gb1_ref, gg1_ref, gB1_ref,
                gW2_ref, gb2_ref, gg2_ref, gB2_ref,
                gW3_ref, gb3_ref,
                fW1a_ref, fW1b_ref, fb1_ref, fg1_ref, fB1_ref,
                fW2_ref, fb2_ref, fg2_ref, fB2_ref,
                fW3_ref, fb3_ref,
                j64_ref, j128_ref,
                out_ref):
    j64 = j64_ref[...]
    j128 = j128_ref[...]
    x = geom_ref[...]
    h = _leaky(_ln(_dot(x, gW1_ref[...]) + gb1_ref[...],
                   gg1_ref[...], gB1_ref[...], j64))
    h = _leaky(_ln(_dot(h, gW2_ref[...]) + gb2_ref[...],
                   gg2_ref[...], gB2_ref[...], j128))
    go = _dot(h, gW3_ref[...]) + gb3_ref[...]

    rel_term = _dot(s_ref[...], fW1b_ref[...])
    a1 = _dot(go, fW1a_ref[...]) + rel_term + fb1_ref[...]
    h = _leaky(_ln(a1, fg1_ref[...], fB1_ref[...], j64))
    h = _leaky(_ln(_dot(h, fW2_ref[...]) + fb2_ref[...],
                   fg2_ref[...], fB2_ref[...], j128))
    ff = _dot(h, fW3_ref[...]) + fb3_ref[...]

    out_ref[...] = jnp.mean(ff.reshape(P, K, D), axis=1)


def _row2(v):
    return v.reshape(1, -1)


def kernel(point_features, geom_features, neighbor_idxs,
           gW1, gb1, gg1, gB1, gW2, gb2, gg2, gB2, gW3, gb3,
           fW1, fb1, fg1, fB1, fW2, fb2, fg2, fB2, fW3, fb3):
    pf = point_features.reshape(N, D)
    fW1a, fW1b = fW1[:D], fW1[D:]

    idx = neighbor_idxs.reshape(-1).astype(jnp.int32)
    pad_rows = (jnp.arange(PAD - NK, dtype=jnp.int32) * 13) % N
    idx = jnp.concatenate([idx, pad_rows]).reshape(1, PAD)
    s = _sc_gather(pf, idx)

    geom = geom_features.reshape(NK, 4)

    wspec = lambda shape: pl.BlockSpec(shape, lambda i: (0, 0))
    in_specs = [
        pl.BlockSpec((R, 4), lambda i: (i, 0)),
        pl.BlockSpec((R, D), lambda i: (i, 0)),
        wspec((4, 64)), wspec((1, 64)), wspec((1, 64)), wspec((1, 64)),
        wspec((64, 128)), wspec((1, 128)), wspec((1, 128)), wspec((1, 128)),
        wspec((128, D)), wspec((1, D)),
        wspec((D, 64)), wspec((D, 64)), wspec((1, 64)), wspec((1, 64)), wspec((1, 64)),
        wspec((64, 128)), wspec((1, 128)), wspec((1, 128)), wspec((1, 128)),
        wspec((128, D)), wspec((1, D)),
        wspec((64, 64)), wspec((128, 128)),
    ]
    out = pl.pallas_call(
        _fused_body,
        grid=(N // P,),
        in_specs=in_specs,
        out_specs=pl.BlockSpec((P, D), lambda i: (i, 0)),
        out_shape=jax.ShapeDtypeStruct((N, D), jnp.float32),
    )(geom, s,
      gW1, _row2(gb1), _row2(gg1), _row2(gB1),
      gW2, _row2(gb2), _row2(gg2), _row2(gB2),
      gW3, _row2(gb3),
      fW1a, fW1b, _row2(fb1), _row2(fg1), _row2(fB1),
      fW2, _row2(fb2), _row2(fg2), _row2(fB2),
      fW3, _row2(fb3),
      jnp.full((64, 64), 1.0 / 64, jnp.float32),
      jnp.full((128, 128), 1.0 / 128, jnp.float32))

    return out.reshape(1, N, D)
